# baseline (speedup 1.0000x reference)
import functools
import math

import jax
import jax.numpy as jnp
from jax.experimental import pallas as pl
from jax.experimental.pallas import tpu as pltpu

NEG_INF = -1e30


def _moe_dense_body(x_ref, wg_ref, bg_ref, w1_ref, b1_ref, w2_ref, b2_ref,
                    out_ref, gate_ref, *, num_experts):
    k = pl.program_id(1)
    xb = x_ref[...]

    @pl.when(k == 0)
    def _compute_gates():
        scores = jnp.dot(xb.astype(jnp.float32), wg_ref[...],
                         preferred_element_type=jnp.float32) + bg_ref[...]
        iota = jax.lax.broadcasted_iota(jnp.int32, scores.shape, 1)
        m0 = jnp.max(scores, axis=-1, keepdims=True)
        i0 = jnp.min(jnp.where(scores == m0, iota, num_experts),
                     axis=-1, keepdims=True)
        masked = jnp.where(iota == i0, NEG_INF, scores)
        m1 = jnp.max(masked, axis=-1, keepdims=True)
        i1 = jnp.min(jnp.where(masked == m1, iota, num_experts),
                     axis=-1, keepdims=True)
        g0 = 1.0 / (1.0 + jnp.exp(m1 - m0))
        gate_ref[...] = (jnp.where(iota == i0, g0, 0.0)
                         + jnp.where(iota == i1, 1.0 - g0, 0.0))

    h = jnp.dot(xb, w1_ref[0],
                preferred_element_type=jnp.float32) + b1_ref[0]
    h = 0.5 * h * (1.0 + jax.lax.erf(h * (1.0 / math.sqrt(2.0))))
    y = jnp.dot(h.astype(jnp.bfloat16), w2_ref[0],
                preferred_element_type=jnp.float32) + b2_ref[0]
    gates = gate_ref[...]
    col = jax.lax.broadcasted_iota(jnp.int32, gates.shape, 1)
    gate_k = jnp.sum(jnp.where(col == k, gates, 0.0), axis=1, keepdims=True)
    contrib = gate_k * y

    @pl.when(k == 0)
    def _init():
        out_ref[...] = contrib

    @pl.when(k != 0)
    def _acc():
        out_ref[...] += contrib


def kernel(x, Wg, bg, W1, b1, W2, b2):
    B, N, EMB = x.shape
    T = B * N
    E, _, HID = W1.shape
    TT = min(2048, T)
    NI = T // TT
    xf = x.reshape(T, EMB).astype(jnp.bfloat16)
    W1 = W1.astype(jnp.bfloat16)
    W2 = W2.astype(jnp.bfloat16)

    out = pl.pallas_call(
        functools.partial(_moe_dense_body, num_experts=E),
        grid=(NI, E),
        in_specs=[
            pl.BlockSpec((TT, EMB), lambda i, k: (i, 0)),
            pl.BlockSpec((EMB, E), lambda i, k: (0, 0)),
            pl.BlockSpec((1, E), lambda i, k: (0, 0)),
            pl.BlockSpec((1, EMB, HID), lambda i, k: (k, 0, 0)),
            pl.BlockSpec((1, 1, HID), lambda i, k: (k, 0, 0)),
            pl.BlockSpec((1, HID, EMB), lambda i, k: (k, 0, 0)),
            pl.BlockSpec((1, 1, EMB), lambda i, k: (k, 0, 0)),
        ],
        out_specs=pl.BlockSpec((TT, EMB), lambda i, k: (i, 0)),
        out_shape=jax.ShapeDtypeStruct((T, EMB), jnp.float32),
        scratch_shapes=[pltpu.VMEM((TT, E), jnp.float32)],
    )(xf, Wg, bg.reshape(1, E), W1, b1.reshape(E, 1, HID), W2,
      b2.reshape(E, 1, EMB))
    return out.reshape(B, N, EMB)


# speedup vs baseline: 1.0554x; 1.0554x over previous
import functools
import math

import jax
import jax.numpy as jnp
from jax.experimental import pallas as pl
from jax.experimental.pallas import tpu as pltpu

NEG_INF = -1e30


def _gate_body(x_ref, wg_ref, bg_ref, gate_ref, xb_ref, *, num_experts):
    xf = x_ref[...]
    scores = jnp.dot(xf, wg_ref[...],
                     preferred_element_type=jnp.float32) + bg_ref[...]
    iota = jax.lax.broadcasted_iota(jnp.int32, scores.shape, 1)
    m0 = jnp.max(scores, axis=-1, keepdims=True)
    i0 = jnp.min(jnp.where(scores == m0, iota, num_experts),
                 axis=-1, keepdims=True)
    masked = jnp.where(iota == i0, NEG_INF, scores)
    m1 = jnp.max(masked, axis=-1, keepdims=True)
    i1 = jnp.min(jnp.where(masked == m1, iota, num_experts),
                 axis=-1, keepdims=True)
    g0 = 1.0 / (1.0 + jnp.exp(m1 - m0))
    gate_ref[...] = (jnp.where(iota == i0, g0, 0.0)
                     + jnp.where(iota == i1, 1.0 - g0, 0.0))
    xb_ref[...] = xf.astype(jnp.bfloat16)


def _ffn_body(xb_ref, gate_in_ref, w1_ref, b1_ref, w2_ref, b2_ref, out_ref):
    k = pl.program_id(0)
    hc = pl.program_id(1)

    h = jnp.dot(xb_ref[...], w1_ref[0].astype(jnp.bfloat16),
                preferred_element_type=jnp.float32) + b1_ref[0]
    h = 0.5 * h * (1.0 + jax.lax.erf(h * (1.0 / math.sqrt(2.0))))
    y = jnp.dot(h.astype(jnp.bfloat16), w2_ref[0].astype(jnp.bfloat16),
                preferred_element_type=jnp.float32)

    gates = gate_in_ref[...]
    col = jax.lax.broadcasted_iota(jnp.int32, gates.shape, 1)
    gate_k = jnp.sum(jnp.where(col == k, gates, 0.0), axis=1, keepdims=True)
    contrib = gate_k * (y + jnp.where(hc == 0, 1.0, 0.0) * b2_ref[0])

    @pl.when((k == 0) & (hc == 0))
    def _init():
        out_ref[...] = contrib

    @pl.when((k != 0) | (hc != 0))
    def _acc():
        out_ref[...] += contrib


def kernel(x, Wg, bg, W1, b1, W2, b2):
    B, N, EMB = x.shape
    T = B * N
    E, _, HID = W1.shape
    HCH = 512 if HID % 512 == 0 else HID
    HC = HID // HCH
    xf = x.reshape(T, EMB)

    gate_full, xb = pl.pallas_call(
        functools.partial(_gate_body, num_experts=E),
        in_specs=[
            pl.BlockSpec((T, EMB), lambda: (0, 0)),
            pl.BlockSpec((EMB, E), lambda: (0, 0)),
            pl.BlockSpec((1, E), lambda: (0, 0)),
        ],
        out_specs=[pl.BlockSpec((T, E), lambda: (0, 0)),
                   pl.BlockSpec((T, EMB), lambda: (0, 0))],
        out_shape=[jax.ShapeDtypeStruct((T, E), jnp.float32),
                   jax.ShapeDtypeStruct((T, EMB), jnp.bfloat16)],
    )(xf, Wg, bg.reshape(1, E))

    out = pl.pallas_call(
        _ffn_body,
        grid=(E, HC),
        in_specs=[
            pl.BlockSpec((T, EMB), lambda k, hc: (0, 0)),
            pl.BlockSpec((T, E), lambda k, hc: (0, 0)),
            pl.BlockSpec((1, EMB, HCH), lambda k, hc: (k, 0, hc)),
            pl.BlockSpec((1, 1, HCH), lambda k, hc: (k, 0, hc)),
            pl.BlockSpec((1, HCH, EMB), lambda k, hc: (k, hc, 0)),
            pl.BlockSpec((1, 1, EMB), lambda k, hc: (k, 0, 0)),
        ],
        out_specs=pl.BlockSpec((T, EMB), lambda k, hc: (0, 0)),
        out_shape=jax.ShapeDtypeStruct((T, EMB), jnp.float32),
    )(xb, gate_full, W1, b1.reshape(E, 1, HID), W2, b2.reshape(E, 1, EMB))
    return out.reshape(B, N, EMB)


# speedup vs baseline: 1.1640x; 1.1029x over previous
import functools
import math

import jax
import jax.numpy as jnp
from jax import lax
from jax.experimental import pallas as pl
from jax.experimental.pallas import tpu as pltpu
from jax.experimental.pallas import tpu_sc as plsc

NEG_INF = -1e30
TILE = 256


def _route_body(x_ref, wg_ref, bg_ref, dest_ref, gates_ref, meta_ref,
                *, num_experts, tile, g_max):
    T = x_ref.shape[0]
    P = 2 * T
    scores = jnp.dot(x_ref[...], wg_ref[...],
                     preferred_element_type=jnp.float32) + bg_ref[...]
    iota = jax.lax.broadcasted_iota(jnp.int32, scores.shape, 1)
    m0 = jnp.max(scores, axis=-1, keepdims=True)
    i0 = jnp.min(jnp.where(scores == m0, iota, num_experts),
                 axis=-1, keepdims=True)
    masked = jnp.where(iota == i0, NEG_INF, scores)
    m1 = jnp.max(masked, axis=-1, keepdims=True)
    i1 = jnp.min(jnp.where(masked == m1, iota, num_experts),
                 axis=-1, keepdims=True)
    g0 = 1.0 / (1.0 + jnp.exp(m1 - m0))

    onehot = jnp.concatenate(
        [(iota == i0).astype(jnp.float32), (iota == i1).astype(jnp.float32)],
        axis=0)

    incl = onehot
    s = 1
    while s < P:
        shifted = jnp.concatenate(
            [jnp.zeros((s, num_experts), jnp.float32), incl[:P - s]], axis=0)
        incl = incl + shifted
        s *= 2
    strict = incl - onehot
    counts = incl[P - 1:P, :]

    counts_i = counts.astype(jnp.int32)
    padded = ((counts_i + (tile - 1)) // tile) * tile
    padded_f = padded.astype(jnp.float32)
    er = jax.lax.broadcasted_iota(jnp.int32, (num_experts, num_experts), 0)
    ec = jax.lax.broadcasted_iota(jnp.int32, (num_experts, num_experts), 1)
    upper = (er < ec).astype(jnp.float32)
    po = jnp.dot(padded_f, upper, preferred_element_type=jnp.float32)
    off_next = po + padded_f

    dest = jnp.sum(onehot * (po + strict), axis=-1, keepdims=True)
    dest_ref[...] = dest.astype(jnp.int32)
    gates_ref[...] = jnp.concatenate([g0, 1.0 - g0], axis=0)

    gi = (jax.lax.broadcasted_iota(jnp.int32, (g_max, num_experts), 0)
          * tile).astype(jnp.float32)
    et = jnp.sum((gi >= off_next).astype(jnp.float32), axis=-1, keepdims=True)
    et = jnp.minimum(et.astype(jnp.int32), num_experts - 1)
    ntiles = (off_next[:, num_experts - 1:] / tile).astype(jnp.int32)
    meta_ref[...] = jnp.concatenate([et, ntiles], axis=0)


def _make_dispatch(T, EMB, NPAD):
    NW = 32
    TPW = T // NW
    CH = 64
    NCH = TPW // CH
    mesh = plsc.VectorSubcoreMesh(core_axis_name="c", subcore_axis_name="s")

    @functools.partial(
        pl.kernel, mesh=mesh,
        out_type=[jax.ShapeDtypeStruct((NPAD, EMB), jnp.float32),
                  jax.ShapeDtypeStruct((NPAD,), jnp.float32)],
        scratch_types=[pltpu.VMEM((CH,), jnp.int32),
                       pltpu.VMEM((CH, EMB), jnp.float32),
                       pltpu.VMEM((CH,), jnp.float32),
                       pltpu.SemaphoreType.DMA,
                       pltpu.SemaphoreType.DMA],
    )
    def dispatch(x_hbm, dest_hbm, gates_hbm, xg_hbm, rg_hbm,
                 idx_v, rows_v, g_v, sem_r, sem_g):
        wid = lax.axis_index("s") * 2 + lax.axis_index("c")
        base = wid * TPW
        for k in range(2):
            for c in range(NCH):
                tok = base + c * CH
                pair = k * T + tok
                pltpu.sync_copy(dest_hbm.at[pl.ds(pair, CH)], idx_v)
                pltpu.sync_copy(gates_hbm.at[pl.ds(pair, CH)], g_v)
                pltpu.sync_copy(x_hbm.at[pl.ds(tok, CH)], rows_v)
                cp_r = pltpu.async_copy(rows_v, xg_hbm.at[idx_v], sem_r)
                cp_g = pltpu.async_copy(g_v, rg_hbm.at[idx_v], sem_g)
                cp_r.wait()
                cp_g.wait()

    return dispatch


def _ffn_body(meta_ref, xg_ref, rg_ref, w1_ref, b1_ref, w2_ref, b2_ref,
              yg_ref, *, g_max):
    g = pl.program_id(0)

    @pl.when(g < meta_ref[g_max])
    def _compute():
        h = jnp.dot(xg_ref[...], w1_ref[0],
                    preferred_element_type=jnp.float32) + b1_ref[0]
        h = 0.5 * h * (1.0 + jax.lax.erf(h * (1.0 / math.sqrt(2.0))))
        y = jnp.dot(h, w2_ref[0],
                    preferred_element_type=jnp.float32) + b2_ref[0]
        yg_ref[...] = y * rg_ref[...]


def _make_combine(T, EMB, NPAD):
    NW = 32
    TPW = T // NW
    CH = 32
    NCH = TPW // CH
    SEG = EMB // 16
    mesh = plsc.VectorSubcoreMesh(core_axis_name="c", subcore_axis_name="s")

    @functools.partial(
        pl.kernel, mesh=mesh,
        out_type=jax.ShapeDtypeStruct((T, EMB), jnp.float32),
        scratch_types=[pltpu.VMEM((CH,), jnp.int32),
                       pltpu.VMEM((CH,), jnp.int32),
                       pltpu.VMEM((CH, EMB), jnp.float32),
                       pltpu.VMEM((CH, EMB), jnp.float32),
                       pltpu.SemaphoreType.DMA,
                       pltpu.SemaphoreType.DMA],
    )
    def combine(yg_hbm, dest_hbm, out_hbm,
                idx0_v, idx1_v, r0_v, r1_v, sem0, sem1):
        wid = lax.axis_index("s") * 2 + lax.axis_index("c")
        base = wid * TPW
        for c in range(NCH):
            tok = base + c * CH
            pltpu.sync_copy(dest_hbm.at[pl.ds(tok, CH)], idx0_v)
            pltpu.sync_copy(dest_hbm.at[pl.ds(T + tok, CH)], idx1_v)
            cp0 = pltpu.async_copy(yg_hbm.at[idx0_v], r0_v, sem0)
            cp1 = pltpu.async_copy(yg_hbm.at[idx1_v], r1_v, sem1)
            cp0.wait()
            cp1.wait()

            def add_body(q, carry):
                i = q // SEG
                j = (q % SEG) * 16
                r0_v[i, pl.ds(j, 16)] = (r0_v[i, pl.ds(j, 16)]
                                         + r1_v[i, pl.ds(j, 16)])
                return carry

            lax.fori_loop(0, CH * SEG, add_body, 0)
            pltpu.sync_copy(r0_v, out_hbm.at[pl.ds(tok, CH)])

    return combine


def kernel(x, Wg, bg, W1, b1, W2, b2):
    B, N, EMB = x.shape
    T = B * N
    E, _, HID = W1.shape
    P = 2 * T
    G_MAX = P // TILE + E
    NPAD = G_MAX * TILE
    xf = x.reshape(T, EMB)

    dest, gates, meta = pl.pallas_call(
        functools.partial(_route_body, num_experts=E, tile=TILE, g_max=G_MAX),
        in_specs=[
            pl.BlockSpec((T, EMB), lambda: (0, 0)),
            pl.BlockSpec((EMB, E), lambda: (0, 0)),
            pl.BlockSpec((1, E), lambda: (0, 0)),
        ],
        out_specs=[pl.BlockSpec((P, 1), lambda: (0, 0)),
                   pl.BlockSpec((P, 1), lambda: (0, 0)),
                   pl.BlockSpec((G_MAX + 1, 1), lambda: (0, 0))],
        out_shape=[jax.ShapeDtypeStruct((P, 1), jnp.int32),
                   jax.ShapeDtypeStruct((P, 1), jnp.float32),
                   jax.ShapeDtypeStruct((G_MAX + 1, 1), jnp.int32)],
    )(xf, Wg, bg.reshape(1, E))

    dest_flat = dest.reshape(P)
    gates_flat = gates.reshape(P)
    meta_flat = meta.reshape(G_MAX + 1)

    xg, rg = _make_dispatch(T, EMB, NPAD)(xf, dest_flat, gates_flat)

    yg = pl.pallas_call(
        functools.partial(_ffn_body, g_max=G_MAX),
        grid_spec=pltpu.PrefetchScalarGridSpec(
            num_scalar_prefetch=1,
            grid=(G_MAX,),
            in_specs=[
                pl.BlockSpec((TILE, EMB), lambda g, m: (g, 0)),
                pl.BlockSpec((TILE, 1), lambda g, m: (g, 0)),
                pl.BlockSpec((1, EMB, HID), lambda g, m: (m[g], 0, 0)),
                pl.BlockSpec((1, 1, HID), lambda g, m: (m[g], 0, 0)),
                pl.BlockSpec((1, HID, EMB), lambda g, m: (m[g], 0, 0)),
                pl.BlockSpec((1, 1, EMB), lambda g, m: (m[g], 0, 0)),
            ],
            out_specs=pl.BlockSpec((TILE, EMB), lambda g, m: (g, 0)),
        ),
        out_shape=jax.ShapeDtypeStruct((NPAD, EMB), jnp.float32),
    )(meta_flat, xg, rg.reshape(NPAD, 1), W1, b1.reshape(E, 1, HID),
      W2, b2.reshape(E, 1, EMB))

    out = _make_combine(T, EMB, NPAD)(yg, dest_flat)
    return out.reshape(B, N, EMB)


# speedup vs baseline: 1.3399x; 1.1511x over previous
import functools
import math

import jax
import jax.numpy as jnp
from jax import lax
from jax.experimental import pallas as pl
from jax.experimental.pallas import tpu as pltpu
from jax.experimental.pallas import tpu_sc as plsc

NEG_INF = -1e30
TILE = 256


def _route_body(x_ref, wg_ref, bg_ref, dest_ref, gates_ref, meta_ref,
                *, num_experts, tile, g_max):
    T = x_ref.shape[0]
    P = 2 * T
    scores = jnp.dot(x_ref[...], wg_ref[...],
                     preferred_element_type=jnp.float32) + bg_ref[...]
    iota = jax.lax.broadcasted_iota(jnp.int32, scores.shape, 1)
    m0 = jnp.max(scores, axis=-1, keepdims=True)
    i0 = jnp.min(jnp.where(scores == m0, iota, num_experts),
                 axis=-1, keepdims=True)
    masked = jnp.where(iota == i0, NEG_INF, scores)
    m1 = jnp.max(masked, axis=-1, keepdims=True)
    i1 = jnp.min(jnp.where(masked == m1, iota, num_experts),
                 axis=-1, keepdims=True)
    g0 = 1.0 / (1.0 + jnp.exp(m1 - m0))

    onehot = jnp.concatenate(
        [(iota == i0).astype(jnp.float32), (iota == i1).astype(jnp.float32)],
        axis=0)

    incl = onehot
    s = 1
    while s < P:
        shifted = jnp.concatenate(
            [jnp.zeros((s, num_experts), jnp.float32), incl[:P - s]], axis=0)
        incl = incl + shifted
        s *= 2
    strict = incl - onehot
    counts = incl[P - 1:P, :]

    counts_i = counts.astype(jnp.int32)
    padded = ((counts_i + (tile - 1)) // tile) * tile
    padded_f = padded.astype(jnp.float32)
    er = jax.lax.broadcasted_iota(jnp.int32, (num_experts, num_experts), 0)
    ec = jax.lax.broadcasted_iota(jnp.int32, (num_experts, num_experts), 1)
    upper = (er < ec).astype(jnp.float32)
    po = jnp.dot(padded_f, upper, preferred_element_type=jnp.float32)
    off_next = po + padded_f

    dest = jnp.sum(onehot * (po + strict), axis=-1, keepdims=True)
    dest_ref[...] = dest.astype(jnp.int32)
    gates_ref[...] = jnp.concatenate([g0, 1.0 - g0], axis=0)

    gi = (jax.lax.broadcasted_iota(jnp.int32, (g_max, num_experts), 0)
          * tile).astype(jnp.float32)
    et = jnp.sum((gi >= off_next).astype(jnp.float32), axis=-1, keepdims=True)
    et = jnp.minimum(et.astype(jnp.int32), num_experts - 1)
    ntiles = (off_next[:, num_experts - 1:] / tile).astype(jnp.int32)
    meta_ref[...] = jnp.concatenate([et, ntiles], axis=0)


def _make_dispatch(T, EMB, NPAD):
    NW = 32
    TPW = T // NW
    CH = 32
    NCH = TPW // CH
    NCHUNK = 2 * NCH
    mesh = plsc.VectorSubcoreMesh(core_axis_name="c", subcore_axis_name="s")

    @functools.partial(
        pl.kernel, mesh=mesh,
        out_type=[jax.ShapeDtypeStruct((NPAD, EMB), jnp.float32),
                  jax.ShapeDtypeStruct((NPAD,), jnp.float32)],
        scratch_types=[pltpu.VMEM((2, NCH, CH), jnp.int32),
                       pltpu.VMEM((2, NCH, CH), jnp.float32),
                       pltpu.VMEM((CH, EMB), jnp.float32),
                       pltpu.VMEM((CH, EMB), jnp.float32),
                       pltpu.SemaphoreType.DMA,
                       pltpu.SemaphoreType.DMA,
                       pltpu.SemaphoreType.DMA,
                       pltpu.SemaphoreType.DMA,
                       pltpu.SemaphoreType.DMA,
                       pltpu.SemaphoreType.DMA],
    )
    def dispatch(x_hbm, dest_hbm, gates_hbm, xg_hbm, rg_hbm,
                 idx_all, g_all, rows_a, rows_b,
                 gs_a, gs_b, ss_a, ss_b, sg_a, sg_b):
        wid = lax.axis_index("s") * 2 + lax.axis_index("c")
        base = wid * TPW
        pltpu.sync_copy(dest_hbm.at[0, wid], idx_all.at[0])
        pltpu.sync_copy(dest_hbm.at[1, wid], idx_all.at[1])
        pltpu.sync_copy(gates_hbm.at[0, wid], g_all.at[0])
        pltpu.sync_copy(gates_hbm.at[1, wid], g_all.at[1])

        bufs = (rows_a, rows_b)
        gsems = (gs_a, gs_b)
        ssems = (ss_a, ss_b)
        gtsems = (sg_a, sg_b)
        chunks = [(k, c) for k in range(2) for c in range(NCH)]

        def gather_in(n):
            _, c = chunks[n]
            tok = base + c * CH
            return pltpu.async_copy(x_hbm.at[pl.ds(tok, CH)],
                                    bufs[n % 2], gsems[n % 2])

        cp_in = {0: gather_in(0), 1: gather_in(1)}
        cp_out = {}
        cp_gt = {}
        for n in range(NCHUNK):
            b = n % 2
            k, c = chunks[n]
            cp_in[n].wait()
            cp_out[n] = pltpu.async_copy(
                bufs[b], xg_hbm.at[idx_all.at[k, c]], ssems[b])
            cp_gt[n] = pltpu.async_copy(
                g_all.at[k, c], rg_hbm.at[idx_all.at[k, c]], gtsems[b])
            if n + 2 < NCHUNK:
                cp_out[n].wait()
                cp_gt[n].wait()
                cp_in[n + 2] = gather_in(n + 2)
        cp_out[NCHUNK - 2].wait()
        cp_gt[NCHUNK - 2].wait()
        cp_out[NCHUNK - 1].wait()
        cp_gt[NCHUNK - 1].wait()

    return dispatch


def _ffn_body(meta_ref, xg_ref, rg_ref, w1_ref, b1_ref, w2_ref, b2_ref,
              yg_ref, *, g_max):
    g = pl.program_id(0)

    @pl.when(g < meta_ref[g_max])
    def _compute():
        h = jnp.dot(xg_ref[...], w1_ref[0],
                    preferred_element_type=jnp.float32) + b1_ref[0]
        h = 0.5 * h * (1.0 + jax.lax.erf(h * (1.0 / math.sqrt(2.0))))
        y = jnp.dot(h, w2_ref[0],
                    preferred_element_type=jnp.float32) + b2_ref[0]
        yg_ref[...] = y * rg_ref[...]


def _make_combine(T, EMB, NPAD):
    NW = 32
    TPW = T // NW
    CH = 16
    NCH = TPW // CH
    SEG = EMB // 16
    UNR = 8
    mesh = plsc.VectorSubcoreMesh(core_axis_name="c", subcore_axis_name="s")

    @functools.partial(
        pl.kernel, mesh=mesh,
        out_type=jax.ShapeDtypeStruct((T, EMB), jnp.float32),
        scratch_types=[pltpu.VMEM((2, NCH, CH), jnp.int32),
                       pltpu.VMEM((CH, EMB), jnp.float32),
                       pltpu.VMEM((CH, EMB), jnp.float32),
                       pltpu.VMEM((CH, EMB), jnp.float32),
                       pltpu.VMEM((CH, EMB), jnp.float32),
                       pltpu.SemaphoreType.DMA,
                       pltpu.SemaphoreType.DMA,
                       pltpu.SemaphoreType.DMA,
                       pltpu.SemaphoreType.DMA,
                       pltpu.SemaphoreType.DMA,
                       pltpu.SemaphoreType.DMA],
    )
    def combine(yg_hbm, dest_hbm, out_hbm,
                idx_all, r0_a, r1_a, r0_b, r1_b,
                s0_a, s1_a, s0_b, s1_b, so_a, so_b):
        wid = lax.axis_index("s") * 2 + lax.axis_index("c")
        base = wid * TPW
        pltpu.sync_copy(dest_hbm.at[0, wid], idx_all.at[0])
        pltpu.sync_copy(dest_hbm.at[1, wid], idx_all.at[1])

        r0s = (r0_a, r0_b)
        r1s = (r1_a, r1_b)
        s0s = (s0_a, s0_b)
        s1s = (s1_a, s1_b)
        sos = (so_a, so_b)

        def gathers(n):
            b = n % 2
            return (pltpu.async_copy(yg_hbm.at[idx_all.at[0, n]],
                                     r0s[b], s0s[b]),
                    pltpu.async_copy(yg_hbm.at[idx_all.at[1, n]],
                                     r1s[b], s1s[b]))

        cp_g = {0: gathers(0), 1: gathers(1)}
        cp_o = {}
        for n in range(NCH):
            b = n % 2
            cp_g[n][0].wait()
            cp_g[n][1].wait()
            r0_v, r1_v = r0s[b], r1s[b]

            def add_body(q, carry, r0_v=r0_v, r1_v=r1_v):
                row = q // (SEG // UNR)
                blk = (q % (SEG // UNR)) * UNR * 16
                for u in range(UNR):
                    j = blk + u * 16
                    r0_v[row, pl.ds(j, 16)] = (r0_v[row, pl.ds(j, 16)]
                                               + r1_v[row, pl.ds(j, 16)])
                return carry

            lax.fori_loop(0, CH * SEG // UNR, add_body, 0)
            tok = base + n * CH
            cp_o[n] = pltpu.async_copy(r0_v, out_hbm.at[pl.ds(tok, CH)],
                                       sos[b])
            if n + 2 < NCH:
                cp_o[n].wait()
                cp_g[n + 2] = gathers(n + 2)
        cp_o[NCH - 2].wait()
        cp_o[NCH - 1].wait()

    return combine


def kernel(x, Wg, bg, W1, b1, W2, b2):
    B, N, EMB = x.shape
    T = B * N
    E, _, HID = W1.shape
    P = 2 * T
    G_MAX = P // TILE + E
    NPAD = G_MAX * TILE
    xf = x.reshape(T, EMB)

    dest, gates, meta = pl.pallas_call(
        functools.partial(_route_body, num_experts=E, tile=TILE, g_max=G_MAX),
        in_specs=[
            pl.BlockSpec((T, EMB), lambda: (0, 0)),
            pl.BlockSpec((EMB, E), lambda: (0, 0)),
            pl.BlockSpec((1, E), lambda: (0, 0)),
        ],
        out_specs=[pl.BlockSpec((P, 1), lambda: (0, 0)),
                   pl.BlockSpec((P, 1), lambda: (0, 0)),
                   pl.BlockSpec((G_MAX + 1, 1), lambda: (0, 0))],
        out_shape=[jax.ShapeDtypeStruct((P, 1), jnp.int32),
                   jax.ShapeDtypeStruct((P, 1), jnp.float32),
                   jax.ShapeDtypeStruct((G_MAX + 1, 1), jnp.int32)],
    )(xf, Wg, bg.reshape(1, E))

    NW = 32
    meta_flat = meta.reshape(G_MAX + 1)
    dest_d = dest.reshape(2, NW, 4, 32)
    gates_d = gates.reshape(2, NW, 4, 32)
    dest_c = dest.reshape(2, NW, 8, 16)

    xg, rg = _make_dispatch(T, EMB, NPAD)(xf, dest_d, gates_d)

    yg = pl.pallas_call(
        functools.partial(_ffn_body, g_max=G_MAX),
        grid_spec=pltpu.PrefetchScalarGridSpec(
            num_scalar_prefetch=1,
            grid=(G_MAX,),
            in_specs=[
                pl.BlockSpec((TILE, EMB), lambda g, m: (g, 0)),
                pl.BlockSpec((TILE, 1), lambda g, m: (g, 0)),
                pl.BlockSpec((1, EMB, HID), lambda g, m: (m[g], 0, 0)),
                pl.BlockSpec((1, 1, HID), lambda g, m: (m[g], 0, 0)),
                pl.BlockSpec((1, HID, EMB), lambda g, m: (m[g], 0, 0)),
                pl.BlockSpec((1, 1, EMB), lambda g, m: (m[g], 0, 0)),
            ],
            out_specs=pl.BlockSpec((TILE, EMB), lambda g, m: (g, 0)),
        ),
        out_shape=jax.ShapeDtypeStruct((NPAD, EMB), jnp.float32),
    )(meta_flat, xg, rg.reshape(NPAD, 1), W1, b1.reshape(E, 1, HID),
      W2, b2.reshape(E, 1, EMB))

    out = _make_combine(T, EMB, NPAD)(yg, dest_c)
    return out.reshape(B, N, EMB)
